# Initial kernel scaffold; baseline (speedup 1.0000x reference)
import jax
import jax.numpy as jnp
from jax.experimental import pallas as pl


def kernel(x_user, x_merchant, edge_index_um, edge_index_mu, emb_user, emb_merchant, l0_um_Wl, l0_um_bl, l0_um_Wr, l0_mu_Wl, l0_mu_bl, l0_mu_Wr, l0_user_g, l0_user_b, l0_mer_g, l0_mer_b, l1_um_Wl, l1_um_bl, l1_um_Wr, l1_mu_Wl, l1_mu_bl, l1_mu_Wr, l1_user_g, l1_user_b, l1_mer_g, l1_mer_b, l2_um_Wl, l2_um_bl, l2_um_Wr, l2_mu_Wl, l2_mu_bl, l2_mu_Wr, l2_user_g, l2_user_b, l2_mer_g, l2_mer_b, cls_W1, cls_b1, cls_W2, cls_b2):
    raise NotImplementedError("write your pallas kernel here")


# speedup vs baseline: 3.1003x; 3.1003x over previous
import functools

import jax
import jax.numpy as jnp
from jax import lax
from jax.experimental import pallas as pl
from jax.experimental.pallas import tpu as pltpu
from jax.experimental.pallas import tpu_sc as plsc

N = 50000
H = 128
NC = 2
NS = 16
NW = NC * NS
NPAD = 50176
R = 12544
R_ACC = R + 8
TPR = R // NS
B = 128
LB = 392
BM = 512
CW = 128
_BN_SCALE = float((1.0 + 1e-5) ** -0.5)


@functools.lru_cache(maxsize=None)
def _mesh():
    return plsc.VectorSubcoreMesh(core_axis_name="c", subcore_axis_name="s",
                                  num_cores=NC, num_subcores=NS)


@functools.lru_cache(maxsize=None)
def _make_seg_sum(width, do_gather):
    @functools.partial(
        pl.kernel,
        out_type=jax.ShapeDtypeStruct((NPAD, width), jnp.float32),
        mesh=_mesh(),
        scratch_types=[
            pltpu.VMEM((16,), jnp.int32),
            pltpu.VMEM((B,), jnp.int32),
            pltpu.VMEM((B,), jnp.int32),
            pltpu.VMEM((B,), jnp.int32),
            pltpu.VMEM((B, width), jnp.float32),
            pltpu.VMEM_SHARED((R_ACC, width), jnp.float32),
        ],
    )
    def seg(table_hbm, srcs_hbm, dsts_hbm, meta_hbm, zrows_hbm, out_hbm,
            meta_v, sidx_v, draw_v, dloc_v, msg_v, acc_sh):
        c = lax.axis_index("c")
        s = lax.axis_index("s")
        w = c * NS + s
        pltpu.sync_copy(meta_hbm.at[pl.ds(w * 16, 16)], meta_v)
        mv = meta_v[...]
        lane = lax.broadcasted_iota(jnp.int32, (16,), 0)

        def ext(j):
            return mv[j]

        if not do_gather:
            pltpu.sync_copy(table_hbm, msg_v)

        for p in range(2):
            chunk = 2 * p + c
            base = chunk * R
            e_lo = ext(2 * p)
            e_hi = ext(2 * p + 1)
            pltpu.sync_copy(zrows_hbm, acc_sh.at[pl.ds(s * TPR, TPR)])

            @pl.when(s == 0)
            def _():
                pltpu.sync_copy(zrows_hbm.at[pl.ds(0, 8)],
                                acc_sh.at[pl.ds(R, 8)])

            plsc.subcore_barrier()

            a_lo = (e_lo // 8) * 8
            nb = jnp.maximum((e_hi - a_lo + B - 1) // B, 0)

            def body(k, _):
                pos = a_lo + k * B
                pltpu.sync_copy(srcs_hbm.at[pl.ds(pos, B)], sidx_v)
                pltpu.sync_copy(dsts_hbm.at[pl.ds(pos, B)], draw_v)
                for j in range(B // 16):
                    d = draw_v[pl.ds(j * 16, 16)]
                    pv = pos + j * 16 + lane
                    ok = (pv >= e_lo) & (pv < e_hi)
                    dloc_v[pl.ds(j * 16, 16)] = jnp.where(
                        ok, d - base, jnp.int32(R))
                if do_gather:
                    pltpu.sync_copy(table_hbm.at[sidx_v], msg_v)
                pltpu.sync_copy(msg_v, acc_sh.at[dloc_v], add=True)
                return 0

            lax.fori_loop(0, nb, body, 0)
            plsc.subcore_barrier()
            pltpu.sync_copy(acc_sh.at[pl.ds(s * TPR, TPR)],
                            out_hbm.at[pl.ds(base + s * TPR, TPR)])

    return seg


@functools.lru_cache(maxsize=None)
def _make_lookup():
    @functools.partial(
        pl.kernel,
        out_type=jax.ShapeDtypeStruct((NPAD, H), jnp.float32),
        mesh=_mesh(),
        scratch_types=[
            pltpu.VMEM((LB,), jnp.int32),
            pltpu.VMEM((LB, H), jnp.float32),
        ],
    )
    def lookup(emb_hbm, idx_hbm, out_hbm, idx_v, rows_v):
        c = lax.axis_index("c")
        s = lax.axis_index("s")
        w = c * NS + s
        for blk in range(NPAD // NW // LB):
            bs = w * (NPAD // NW) + blk * LB
            pltpu.sync_copy(idx_hbm.at[pl.ds(bs, LB)], idx_v)
            pltpu.sync_copy(emb_hbm.at[idx_v], rows_v)
            pltpu.sync_copy(rows_v, out_hbm.at[pl.ds(bs, LB)])

    return lookup


def _combine_body(acc_ref, cnt_ref, h_ref, wl_ref, wr_ref, bl_ref, g_ref,
                  b_ref, o_ref):
    cnt = cnt_ref[:, 0:1]
    mean = acc_ref[...] * (1.0 / jnp.maximum(cnt, 1.0))
    o = jnp.dot(mean, wl_ref[...], preferred_element_type=jnp.float32)
    o = o + jnp.dot(h_ref[...], wr_ref[...], preferred_element_type=jnp.float32)
    o = o + bl_ref[...]
    o_ref[...] = jnp.maximum(o * g_ref[...] + b_ref[...], 0.0)


def _combine(acc, cnt, h, wlT, wrT, bl, g_eff, b):
    return pl.pallas_call(
        _combine_body,
        grid=(NPAD // BM,),
        in_specs=[
            pl.BlockSpec((BM, H), lambda i: (i, 0)),
            pl.BlockSpec((BM, CW), lambda i: (i, 0)),
            pl.BlockSpec((BM, H), lambda i: (i, 0)),
            pl.BlockSpec((H, H), lambda i: (0, 0)),
            pl.BlockSpec((H, H), lambda i: (0, 0)),
            pl.BlockSpec((1, H), lambda i: (0, 0)),
            pl.BlockSpec((1, H), lambda i: (0, 0)),
            pl.BlockSpec((1, H), lambda i: (0, 0)),
        ],
        out_specs=pl.BlockSpec((BM, H), lambda i: (i, 0)),
        out_shape=jax.ShapeDtypeStruct((NPAD, H), jnp.float32),
    )(acc, cnt, h, wlT, wrT, bl, g_eff, b)


def _cls_body(h_ref, w1_ref, b1_ref, w2_ref, b2_ref, o_ref):
    z = jnp.dot(h_ref[...], w1_ref[...], preferred_element_type=jnp.float32)
    z = jnp.maximum(z + b1_ref[...], 0.0)
    o = jnp.dot(z, w2_ref[...], preferred_element_type=jnp.float32)
    o_ref[...] = o + b2_ref[...]


def _classifier(h, w1T, b1, w2Tp, b2p):
    return pl.pallas_call(
        _cls_body,
        grid=(NPAD // BM,),
        in_specs=[
            pl.BlockSpec((BM, H), lambda i: (i, 0)),
            pl.BlockSpec((H, H // 2), lambda i: (0, 0)),
            pl.BlockSpec((1, H // 2), lambda i: (0, 0)),
            pl.BlockSpec((H // 2, H), lambda i: (0, 0)),
            pl.BlockSpec((1, H), lambda i: (0, 0)),
        ],
        out_specs=pl.BlockSpec((BM, H), lambda i: (i, 0)),
        out_shape=jax.ShapeDtypeStruct((NPAD, H), jnp.float32),
    )(h, w1T, b1, w2Tp, b2p)


def _prep_edges(src, dst):
    src = src.astype(jnp.uint32)
    dst = dst.astype(jnp.uint32)
    key = jnp.left_shift(dst, jnp.uint32(16)) | src
    key_s = jnp.sort(key)
    dst_s = jnp.right_shift(key_s, jnp.uint32(16)).astype(jnp.int32)
    src_s = (key_s & jnp.uint32(0xFFFF)).astype(jnp.int32)
    pad_src = (jnp.arange(B, dtype=jnp.int32) * 397) % N
    srcs = jnp.concatenate([src_s, pad_src])
    dsts = jnp.concatenate([dst_s, jnp.zeros((B,), jnp.int32)])
    bounds = jnp.arange(5, dtype=jnp.int32) * R
    cuts = jnp.searchsorted(dst_s, bounds, side="left").astype(jnp.int32)
    lo = cuts[:4]
    hi = cuts[1:]
    per = (hi - lo + NS - 1) // NS
    t = jnp.arange(NS, dtype=jnp.int32)
    t_lo = jnp.minimum(lo[:, None] + t[None, :] * per[:, None], hi[:, None])
    t_hi = jnp.minimum(t_lo + per[:, None], hi[:, None])
    rows = []
    for c in range(NC):
        lanes = [t_lo[c], t_hi[c], t_lo[c + 2], t_hi[c + 2]]
        lanes += [jnp.zeros((NS,), jnp.int32)] * 12
        rows.append(jnp.stack(lanes, axis=-1))
    meta = jnp.stack(rows).reshape(-1).astype(jnp.int32)
    return srcs, dsts, meta


def _pad_idx(x, vocab):
    extra = NPAD - x.shape[0]
    tail = (jnp.arange(extra, dtype=jnp.int32) * 13) % vocab
    return jnp.concatenate([x.astype(jnp.int32), tail])


def kernel(x_user, x_merchant, edge_index_um, edge_index_mu, emb_user,
           emb_merchant, l0_um_Wl, l0_um_bl, l0_um_Wr, l0_mu_Wl, l0_mu_bl,
           l0_mu_Wr, l0_user_g, l0_user_b, l0_mer_g, l0_mer_b, l1_um_Wl,
           l1_um_bl, l1_um_Wr, l1_mu_Wl, l1_mu_bl, l1_mu_Wr, l1_user_g,
           l1_user_b, l1_mer_g, l1_mer_b, l2_um_Wl, l2_um_bl, l2_um_Wr,
           l2_mu_Wl, l2_mu_bl, l2_mu_Wr, l2_user_g, l2_user_b, l2_mer_g,
           l2_mer_b, cls_W1, cls_b1, cls_W2, cls_b2):
    params = {
        0: (l0_um_Wl, l0_um_bl, l0_um_Wr, l0_mu_Wl, l0_mu_bl, l0_mu_Wr,
            l0_user_g, l0_user_b, l0_mer_g, l0_mer_b),
        1: (l1_um_Wl, l1_um_bl, l1_um_Wr, l1_mu_Wl, l1_mu_bl, l1_mu_Wr,
            l1_user_g, l1_user_b, l1_mer_g, l1_mer_b),
        2: (l2_um_Wl, l2_um_bl, l2_um_Wr, l2_mu_Wl, l2_mu_bl, l2_mu_Wr,
            l2_user_g, l2_user_b, l2_mer_g, l2_mer_b),
    }

    lookup = _make_lookup()
    seg_feat = _make_seg_sum(H, True)
    seg_cnt = _make_seg_sum(CW, False)

    xu = _pad_idx(x_user, emb_user.shape[0])
    xm = _pad_idx(x_merchant, emb_merchant.shape[0])
    h_u = lookup(emb_user, xu)
    h_m = lookup(emb_merchant, xm)

    srcs_um, dsts_um, meta_um = _prep_edges(edge_index_um[0],
                                            edge_index_um[1])
    srcs_mu, dsts_mu, meta_mu = _prep_edges(edge_index_mu[0],
                                            edge_index_mu[1])

    ones_blk = jnp.ones((B, CW), jnp.float32)
    z_feat = jnp.zeros((TPR, H), jnp.float32)
    z_cnt = jnp.zeros((TPR, CW), jnp.float32)

    cnt_m = seg_cnt(ones_blk, srcs_um, dsts_um, meta_um, z_cnt)
    cnt_u = seg_cnt(ones_blk, srcs_mu, dsts_mu, meta_mu, z_cnt)

    for l in range(3):
        (um_Wl, um_bl, um_Wr, mu_Wl, mu_bl, mu_Wr,
         user_g, user_b, mer_g, mer_b) = params[l]
        agg_m = seg_feat(h_u, srcs_um, dsts_um, meta_um, z_feat)
        agg_u = seg_feat(h_m, srcs_mu, dsts_mu, meta_mu, z_feat)
        h_m_new = _combine(agg_m, cnt_m, h_m, um_Wl.T, um_Wr.T,
                           um_bl.reshape(1, H),
                           (mer_g * _BN_SCALE).reshape(1, H),
                           mer_b.reshape(1, H))
        h_u_new = _combine(agg_u, cnt_u, h_u, mu_Wl.T, mu_Wr.T,
                           mu_bl.reshape(1, H),
                           (user_g * _BN_SCALE).reshape(1, H),
                           user_b.reshape(1, H))
        h_u, h_m = h_u_new, h_m_new

    w1T = cls_W1.T
    b1 = cls_b1.reshape(1, H // 2)
    w2Tp = jnp.pad(cls_W2.T, ((0, 0), (0, H - cls_W2.shape[0])))
    b2p = jnp.pad(cls_b2.reshape(1, -1), ((0, 0), (0, H - cls_b2.shape[0])))
    out_u = _classifier(h_u, w1T, b1, w2Tp, b2p)
    out_m = _classifier(h_m, w1T, b1, w2Tp, b2p)
    return jnp.concatenate([out_u[:N, :2], out_m[:N, :2]], axis=0)


# speedup vs baseline: 3.9123x; 1.2619x over previous
import functools

import jax
import jax.numpy as jnp
from jax import lax
from jax.experimental import pallas as pl
from jax.experimental.pallas import tpu as pltpu
from jax.experimental.pallas import tpu_sc as plsc

N = 50000
H = 128
NC = 2
NS = 16
NW = NC * NS
NPAD = 50176
NP = 4
R = 6272
R_ACC = R + 8
TPR = R // NS
B = 128
LB = 392
BM = 512
CW = 128
_BN_SCALE = float((1.0 + 1e-5) ** -0.5)


@functools.lru_cache(maxsize=None)
def _mesh():
    return plsc.VectorSubcoreMesh(core_axis_name="c", subcore_axis_name="s",
                                  num_cores=NC, num_subcores=NS)


@functools.lru_cache(maxsize=None)
def _make_seg_sum(width, do_gather):
    @functools.partial(
        pl.kernel,
        out_type=jax.ShapeDtypeStruct((NPAD, width), jnp.float32),
        mesh=_mesh(),
        scratch_types=[
            pltpu.VMEM((16,), jnp.int32),
            pltpu.VMEM((B,), jnp.uint32),
            pltpu.VMEM((B,), jnp.uint32),
            pltpu.VMEM((B,), jnp.int32),
            pltpu.VMEM((B,), jnp.int32),
            pltpu.VMEM((B,), jnp.int32),
            pltpu.VMEM((B,), jnp.int32),
            pltpu.VMEM((B, width), jnp.float32),
            pltpu.VMEM((B, width), jnp.float32),
            pltpu.VMEM_SHARED((R_ACC, width), jnp.float32),
            pltpu.SemaphoreType.DMA,
            pltpu.SemaphoreType.DMA,
            pltpu.SemaphoreType.DMA,
            pltpu.SemaphoreType.DMA,
        ],
    )
    def seg(table_hbm, keys_hbm, meta_hbm, zrows_hbm, out_hbm,
            meta_v, key_v0, key_v1, sidx_v0, sidx_v1, dloc_v0, dloc_v1,
            msg0, msg1, acc_sh, sem_k0, sem_k1, sem_s0, sem_s1):
        c = lax.axis_index("c")
        s = lax.axis_index("s")
        w = c * NS + s
        pltpu.sync_copy(meta_hbm.at[pl.ds(w * 16, 16)], meta_v)
        mv = meta_v[...]
        lane = lax.broadcasted_iota(jnp.int32, (16,), 0)

        key_v = (key_v0, key_v1)
        sidx_v = (sidx_v0, sidx_v1)
        dloc_v = (dloc_v0, dloc_v1)
        msg = (msg0, msg1)
        sem_k = (sem_k0, sem_k1)
        sem_s = (sem_s0, sem_s1)

        if not do_gather:
            pltpu.sync_copy(table_hbm, msg0)
            pltpu.sync_copy(table_hbm, msg1)

        for p in range(NP):
            chunk = 2 * p + c
            base = chunk * R
            e_lo = mv[2 * p]
            e_hi = mv[2 * p + 1]
            pltpu.sync_copy(zrows_hbm, acc_sh.at[pl.ds(s * TPR, TPR)])

            @pl.when(s == 0)
            def _():
                pltpu.sync_copy(zrows_hbm.at[pl.ds(0, 8)],
                                acc_sh.at[pl.ds(R, 8)])

            plsc.subcore_barrier()

            a_lo = (e_lo // 8) * 8
            nb = jnp.maximum((e_hi - a_lo + B - 1) // B, 0)
            nbp = (nb + 1) // 2

            @pl.when(nbp > 0)
            def _():
                pltpu.async_copy(keys_hbm.at[pl.ds(a_lo, B)], key_v0, sem_k0)
                pltpu.async_copy(keys_hbm.at[pl.ds(a_lo + B, B)], key_v1,
                                 sem_k1)

            def body(kk, _):
                for sub in range(2):
                    k = 2 * kk + sub
                    pos = a_lo + k * B
                    pltpu.make_async_copy(keys_hbm.at[pl.ds(pos, B)],
                                          key_v[sub], sem_k[sub]).wait()

                    @pl.when(kk >= 1)
                    def _():
                        pltpu.make_async_copy(
                            msg[sub], acc_sh.at[dloc_v[sub]],
                            sem_s[sub]).wait()

                    for j in range(B // 16):
                        kv = key_v[sub][pl.ds(j * 16, 16)]
                        d = jnp.right_shift(kv, jnp.uint32(16)).astype(
                            jnp.int32)
                        si = (kv & jnp.uint32(0xFFFF)).astype(jnp.int32)
                        pv = pos + j * 16 + lane
                        ok = (pv >= e_lo) & (pv < e_hi)
                        dloc_v[sub][pl.ds(j * 16, 16)] = jnp.where(
                            ok, d - base, jnp.int32(R))
                        sidx_v[sub][pl.ds(j * 16, 16)] = si

                    @pl.when(kk + 1 < nbp)
                    def _():
                        pltpu.async_copy(
                            keys_hbm.at[pl.ds(pos + 2 * B, B)],
                            key_v[sub], sem_k[sub])

                    if do_gather:
                        pltpu.sync_copy(table_hbm.at[sidx_v[sub]], msg[sub])
                    pltpu.async_copy(msg[sub], acc_sh.at[dloc_v[sub]],
                                     sem_s[sub], add=True)
                return 0

            lax.fori_loop(0, nbp, body, 0)

            @pl.when(nbp > 0)
            def _():
                pltpu.make_async_copy(msg0, acc_sh.at[dloc_v0],
                                      sem_s0).wait()
                pltpu.make_async_copy(msg1, acc_sh.at[dloc_v1],
                                      sem_s1).wait()

            plsc.subcore_barrier()
            pltpu.sync_copy(acc_sh.at[pl.ds(s * TPR, TPR)],
                            out_hbm.at[pl.ds(base + s * TPR, TPR)])

    return seg


@functools.lru_cache(maxsize=None)
def _make_lookup():
    nblk = NPAD // NW // LB

    @functools.partial(
        pl.kernel,
        out_type=jax.ShapeDtypeStruct((NPAD, H), jnp.float32),
        mesh=_mesh(),
        scratch_types=[
            pltpu.VMEM((LB,), jnp.int32),
            pltpu.VMEM((LB,), jnp.int32),
            pltpu.VMEM((LB, H), jnp.float32),
            pltpu.VMEM((LB, H), jnp.float32),
            pltpu.SemaphoreType.DMA,
            pltpu.SemaphoreType.DMA,
            pltpu.SemaphoreType.DMA,
            pltpu.SemaphoreType.DMA,
        ],
    )
    def lookup(emb_hbm, idx_hbm, out_hbm, idx_v0, idx_v1, rows_v0, rows_v1,
               sem_i0, sem_i1, sem_w0, sem_w1):
        c = lax.axis_index("c")
        s = lax.axis_index("s")
        w = c * NS + s
        idx_v = (idx_v0, idx_v1)
        rows_v = (rows_v0, rows_v1)
        sem_i = (sem_i0, sem_i1)
        sem_w = (sem_w0, sem_w1)
        base = w * (NPAD // NW)
        pltpu.async_copy(idx_hbm.at[pl.ds(base, LB)], idx_v0, sem_i0)
        for blk in range(nblk):
            b = blk % 2
            bs = base + blk * LB
            pltpu.make_async_copy(idx_hbm.at[pl.ds(bs, LB)], idx_v[b],
                                  sem_i[b]).wait()
            if blk + 1 < nblk:
                pltpu.async_copy(idx_hbm.at[pl.ds(bs + LB, LB)],
                                 idx_v[1 - b], sem_i[1 - b])
            if blk >= 2:
                pltpu.make_async_copy(rows_v[b],
                                      out_hbm.at[pl.ds(bs - 2 * LB, LB)],
                                      sem_w[b]).wait()
            pltpu.sync_copy(emb_hbm.at[idx_v[b]], rows_v[b])
            pltpu.async_copy(rows_v[b], out_hbm.at[pl.ds(bs, LB)], sem_w[b])
        for blk in range(max(nblk - 2, 0), nblk):
            b = blk % 2
            bs = base + blk * LB
            pltpu.make_async_copy(rows_v[b], out_hbm.at[pl.ds(bs, LB)],
                                  sem_w[b]).wait()

    return lookup


def _combine_body(acc_ref, cnt_ref, h_ref, wl_ref, wr_ref, bl_ref, g_ref,
                  b_ref, o_ref):
    cnt = cnt_ref[:, 0:1]
    mean = acc_ref[...] * (1.0 / jnp.maximum(cnt, 1.0))
    o = jnp.dot(mean, wl_ref[...], preferred_element_type=jnp.float32)
    o = o + jnp.dot(h_ref[...], wr_ref[...], preferred_element_type=jnp.float32)
    o = o + bl_ref[...]
    o_ref[...] = jnp.maximum(o * g_ref[...] + b_ref[...], 0.0)


def _combine(acc, cnt, h, wlT, wrT, bl, g_eff, b):
    return pl.pallas_call(
        _combine_body,
        grid=(NPAD // BM,),
        in_specs=[
            pl.BlockSpec((BM, H), lambda i: (i, 0)),
            pl.BlockSpec((BM, CW), lambda i: (i, 0)),
            pl.BlockSpec((BM, H), lambda i: (i, 0)),
            pl.BlockSpec((H, H), lambda i: (0, 0)),
            pl.BlockSpec((H, H), lambda i: (0, 0)),
            pl.BlockSpec((1, H), lambda i: (0, 0)),
            pl.BlockSpec((1, H), lambda i: (0, 0)),
            pl.BlockSpec((1, H), lambda i: (0, 0)),
        ],
        out_specs=pl.BlockSpec((BM, H), lambda i: (i, 0)),
        out_shape=jax.ShapeDtypeStruct((NPAD, H), jnp.float32),
    )(acc, cnt, h, wlT, wrT, bl, g_eff, b)


def _cls_body(h_ref, w1_ref, b1_ref, w2_ref, b2_ref, o_ref):
    z = jnp.dot(h_ref[...], w1_ref[...], preferred_element_type=jnp.float32)
    z = jnp.maximum(z + b1_ref[...], 0.0)
    o = jnp.dot(z, w2_ref[...], preferred_element_type=jnp.float32)
    o_ref[...] = o + b2_ref[...]


def _classifier(h, w1T, b1, w2Tp, b2p):
    return pl.pallas_call(
        _cls_body,
        grid=(NPAD // BM,),
        in_specs=[
            pl.BlockSpec((BM, H), lambda i: (i, 0)),
            pl.BlockSpec((H, H // 2), lambda i: (0, 0)),
            pl.BlockSpec((1, H // 2), lambda i: (0, 0)),
            pl.BlockSpec((H // 2, H), lambda i: (0, 0)),
            pl.BlockSpec((1, H), lambda i: (0, 0)),
        ],
        out_specs=pl.BlockSpec((BM, H), lambda i: (i, 0)),
        out_shape=jax.ShapeDtypeStruct((NPAD, H), jnp.float32),
    )(h, w1T, b1, w2Tp, b2p)


def _prep_edges(src, dst):
    src = src.astype(jnp.uint32)
    dst = dst.astype(jnp.uint32)
    key = jnp.left_shift(dst, jnp.uint32(16)) | src
    key_s = jnp.sort(key)
    dst_s = jnp.right_shift(key_s, jnp.uint32(16)).astype(jnp.int32)
    pad_key = ((jnp.arange(2 * B, dtype=jnp.int32) * 397) % N).astype(
        jnp.uint32)
    keys = jnp.concatenate([key_s, pad_key])
    nchunk = NPAD // R
    bounds = jnp.arange(nchunk + 1, dtype=jnp.int32) * R
    cuts = jnp.searchsorted(dst_s, bounds, side="left").astype(jnp.int32)
    lo = cuts[:nchunk]
    hi = cuts[1:]
    per = (hi - lo + NS - 1) // NS
    t = jnp.arange(NS, dtype=jnp.int32)
    t_lo = jnp.minimum(lo[:, None] + t[None, :] * per[:, None], hi[:, None])
    t_hi = jnp.minimum(t_lo + per[:, None], hi[:, None])
    rows = []
    for c in range(NC):
        lanes = []
        for pp in range(NP):
            lanes += [t_lo[2 * pp + c], t_hi[2 * pp + c]]
        lanes += [jnp.zeros((NS,), jnp.int32)] * (16 - 2 * NP)
        rows.append(jnp.stack(lanes, axis=-1))
    meta = jnp.stack(rows).reshape(-1).astype(jnp.int32)
    return keys, meta


def _pad_idx(x, vocab):
    extra = NPAD - x.shape[0]
    tail = (jnp.arange(extra, dtype=jnp.int32) * 13) % vocab
    return jnp.concatenate([x.astype(jnp.int32), tail])


def kernel(x_user, x_merchant, edge_index_um, edge_index_mu, emb_user,
           emb_merchant, l0_um_Wl, l0_um_bl, l0_um_Wr, l0_mu_Wl, l0_mu_bl,
           l0_mu_Wr, l0_user_g, l0_user_b, l0_mer_g, l0_mer_b, l1_um_Wl,
           l1_um_bl, l1_um_Wr, l1_mu_Wl, l1_mu_bl, l1_mu_Wr, l1_user_g,
           l1_user_b, l1_mer_g, l1_mer_b, l2_um_Wl, l2_um_bl, l2_um_Wr,
           l2_mu_Wl, l2_mu_bl, l2_mu_Wr, l2_user_g, l2_user_b, l2_mer_g,
           l2_mer_b, cls_W1, cls_b1, cls_W2, cls_b2):
    params = {
        0: (l0_um_Wl, l0_um_bl, l0_um_Wr, l0_mu_Wl, l0_mu_bl, l0_mu_Wr,
            l0_user_g, l0_user_b, l0_mer_g, l0_mer_b),
        1: (l1_um_Wl, l1_um_bl, l1_um_Wr, l1_mu_Wl, l1_mu_bl, l1_mu_Wr,
            l1_user_g, l1_user_b, l1_mer_g, l1_mer_b),
        2: (l2_um_Wl, l2_um_bl, l2_um_Wr, l2_mu_Wl, l2_mu_bl, l2_mu_Wr,
            l2_user_g, l2_user_b, l2_mer_g, l2_mer_b),
    }

    lookup = _make_lookup()
    seg_feat = _make_seg_sum(H, True)
    seg_cnt = _make_seg_sum(CW, False)

    xu = _pad_idx(x_user, emb_user.shape[0])
    xm = _pad_idx(x_merchant, emb_merchant.shape[0])
    h_u = lookup(emb_user, xu)
    h_m = lookup(emb_merchant, xm)

    keys_um, meta_um = _prep_edges(edge_index_um[0], edge_index_um[1])
    keys_mu, meta_mu = _prep_edges(edge_index_mu[0], edge_index_mu[1])

    ones_blk = jnp.ones((B, CW), jnp.float32)
    z_feat = jnp.zeros((TPR, H), jnp.float32)
    z_cnt = jnp.zeros((TPR, CW), jnp.float32)

    cnt_m = seg_cnt(ones_blk, keys_um, meta_um, z_cnt)
    cnt_u = seg_cnt(ones_blk, keys_mu, meta_mu, z_cnt)

    for l in range(3):
        (um_Wl, um_bl, um_Wr, mu_Wl, mu_bl, mu_Wr,
         user_g, user_b, mer_g, mer_b) = params[l]
        agg_m = seg_feat(h_u, keys_um, meta_um, z_feat)
        agg_u = seg_feat(h_m, keys_mu, meta_mu, z_feat)
        h_m_new = _combine(agg_m, cnt_m, h_m, um_Wl.T, um_Wr.T,
                           um_bl.reshape(1, H),
                           (mer_g * _BN_SCALE).reshape(1, H),
                           mer_b.reshape(1, H))
        h_u_new = _combine(agg_u, cnt_u, h_u, mu_Wl.T, mu_Wr.T,
                           mu_bl.reshape(1, H),
                           (user_g * _BN_SCALE).reshape(1, H),
                           user_b.reshape(1, H))
        h_u, h_m = h_u_new, h_m_new

    w1T = cls_W1.T
    b1 = cls_b1.reshape(1, H // 2)
    w2Tp = jnp.pad(cls_W2.T, ((0, 0), (0, H - cls_W2.shape[0])))
    b2p = jnp.pad(cls_b2.reshape(1, -1), ((0, 0), (0, H - cls_b2.shape[0])))
    out_u = _classifier(h_u, w1T, b1, w2Tp, b2p)
    out_m = _classifier(h_m, w1T, b1, w2Tp, b2p)
    return jnp.concatenate([out_u[:N, :2], out_m[:N, :2]], axis=0)
